# baseline (speedup 1.0000x reference)
import functools

import jax
import jax.numpy as jnp
from jax import lax
from jax.experimental import pallas as pl
from jax.experimental.pallas import tpu as pltpu

NUM_EMB = 8192
DIM = 256
TOKENS = 16384
MB = 256
NB = 1024
NM = TOKENS // MB
NN = NUM_EMB // NB

NW = 32
BPW = TOKENS // NW


def _argmin_body(x_ref, e_ref, idx_ref, dmin_ref, cnt_ref, rmin, ridx):
    j = pl.program_id(1)

    @pl.when(j == 0)
    def _():
        rmin[...] = jnp.full((MB, 1), jnp.inf, jnp.float32)
        ridx[...] = jnp.zeros((MB, 1), jnp.int32)

    x = x_ref[...]
    e = e_ref[...]
    xsq = jnp.sum(x * x, axis=1, keepdims=True)
    esq = jnp.sum(e * e, axis=1).reshape(1, NB)
    mm = lax.dot_general(x, e, (((1,), (1,)), ((), ())),
                         preferred_element_type=jnp.float32)
    dist = (xsq + esq) - 2.0 * mm

    tmin = jnp.min(dist, axis=1, keepdims=True)
    cidx = lax.broadcasted_iota(jnp.int32, (MB, NB), 1) + j * NB
    targ = jnp.min(jnp.where(dist == tmin, cidx, jnp.int32(2**30)),
                   axis=1, keepdims=True)
    better = tmin < rmin[...]
    ridx[...] = jnp.where(better, targ, ridx[...])
    rmin[...] = jnp.where(better, tmin, rmin[...])

    @pl.when(j == NN - 1)
    def _():
        idx_ref[0, 0, :] = ridx[:, 0]
        dmin_ref[0, 0, :] = rmin[:, 0]
        bins = lax.broadcasted_iota(jnp.int32, (MB, NUM_EMB), 1)
        onehot = jnp.where(ridx[...] == bins, jnp.int32(1), jnp.int32(0))
        cnt_ref[0] = jnp.sum(onehot, axis=0, keepdims=True)


_argmin_call = pl.pallas_call(
    _argmin_body,
    grid=(NM, NN),
    in_specs=[
        pl.BlockSpec((MB, DIM), lambda i, j: (i, 0)),
        pl.BlockSpec((NB, DIM), lambda i, j: (j, 0)),
    ],
    out_specs=[
        pl.BlockSpec((1, 1, MB), lambda i, j: (i, 0, 0)),
        pl.BlockSpec((1, 1, MB), lambda i, j: (i, 0, 0)),
        pl.BlockSpec((1, 1, NUM_EMB), lambda i, j: (i, 0, 0)),
    ],
    out_shape=[
        jax.ShapeDtypeStruct((NM, 1, MB), jnp.int32),
        jax.ShapeDtypeStruct((NM, 1, MB), jnp.float32),
        jax.ShapeDtypeStruct((NM, 1, NUM_EMB), jnp.int32),
    ],
    scratch_shapes=[
        pltpu.VMEM((MB, 1), jnp.float32),
        pltpu.VMEM((MB, 1), jnp.int32),
    ],
    compiler_params=pltpu.CompilerParams(
        dimension_semantics=("parallel", "arbitrary")),
)


@functools.lru_cache(maxsize=1)
def _sc_gather_call():
    from jax.experimental.pallas import tpu_sc as plsc

    mesh = plsc.VectorSubcoreMesh(core_axis_name="c", subcore_axis_name="s")

    @functools.partial(
        pl.kernel, mesh=mesh,
        out_type=jax.ShapeDtypeStruct((TOKENS, DIM), jnp.float32),
        scratch_types=[
            pltpu.VMEM((256,), jnp.int32),
            pltpu.VMEM((256, DIM), jnp.float32),
            pltpu.SemaphoreType.DMA,
        ],
    )
    def sc_body(w_hbm, idx_hbm, quant_hbm, idx_g, rows_v, sem):
        c = lax.axis_index("c")
        s = lax.axis_index("s")
        wid = s * 2 + c
        base = wid * BPW

        for chunk in range(2):
            off = base + chunk * 256
            pltpu.sync_copy(idx_hbm.at[pl.ds(off, 256)], idx_g)
            pltpu.async_copy(w_hbm.at[idx_g], rows_v, sem).wait()
            pltpu.sync_copy(rows_v, quant_hbm.at[pl.ds(off, 256)])

    return sc_body


def _scalar_body(dmin_ref, counts_ref, loss_ref, perp_ref):
    dmin = dmin_ref[...]
    loss_ref[0, 0] = 1.25 * jnp.sum(dmin) / (float(TOKENS) * float(DIM))
    csum = jnp.sum(counts_ref[...], axis=0)
    avg = csum.astype(jnp.float32) * (1.0 / float(TOKENS))
    plog = avg * jnp.log(avg + 1e-10)
    perp_ref[0, 0] = jnp.exp(-jnp.sum(plog))


_scalar_call = pl.pallas_call(
    _scalar_body,
    in_specs=[
        pl.BlockSpec((NM, 1, MB), lambda: (0, 0, 0)),
        pl.BlockSpec((NM, 1, NUM_EMB), lambda: (0, 0, 0)),
    ],
    out_specs=[
        pl.BlockSpec(memory_space=pltpu.SMEM),
        pl.BlockSpec(memory_space=pltpu.SMEM),
    ],
    out_shape=[
        jax.ShapeDtypeStruct((1, 1), jnp.float32),
        jax.ShapeDtypeStruct((1, 1), jnp.float32),
    ],
)


def kernel(inputs, embedding_weight):
    input_shape = inputs.shape
    flat = inputs.reshape(-1, DIM)
    idx3, dmin3, cnt3 = _argmin_call(flat, embedding_weight)
    idx = idx3.reshape(-1)

    quant = _sc_gather_call()(embedding_weight, idx)
    loss2, perp2 = _scalar_call(dmin3, cnt3)

    quant = quant.reshape(input_shape)
    quantized_st = inputs + (quant - inputs)
    return (loss2[0, 0], quantized_st, perp2[0, 0],
            idx.reshape(input_shape[0], -1))


# speedup vs baseline: 1.0095x; 1.0095x over previous
import functools

import jax
import jax.numpy as jnp
from jax import lax
from jax.experimental import pallas as pl
from jax.experimental.pallas import tpu as pltpu

NUM_EMB = 8192
DIM = 256
TOKENS = 16384
MB = 256
NB = 1024
NM = TOKENS // MB
NN = NUM_EMB // NB

NW = 32
BPW = TOKENS // NW


def _esq_body(e_ref, esq_ref):
    e = e_ref[...]
    esq_ref[...] = jnp.sum(e * e, axis=1).reshape(1, NB)


_esq_call = pl.pallas_call(
    _esq_body,
    grid=(NN,),
    in_specs=[pl.BlockSpec((NB, DIM), lambda j: (j, 0))],
    out_specs=pl.BlockSpec((1, NB), lambda j: (0, j)),
    out_shape=jax.ShapeDtypeStruct((1, NUM_EMB), jnp.float32),
)


def _argmin_body(x_ref, e_ref, esq_ref, idx_ref, dmin_ref, cnt_ref, rmax, ridx):
    j = pl.program_id(1)

    @pl.when(j == 0)
    def _():
        rmax[...] = jnp.full((MB, 1), -jnp.inf, jnp.float32)
        ridx[...] = jnp.zeros((MB, 1), jnp.int32)

    x = x_ref[...]
    e = e_ref[...]

    x2 = (x + x).astype(jnp.bfloat16)
    mm2 = lax.dot_general(x2, e.astype(jnp.bfloat16), (((1,), (1,)), ((), ())),
                          preferred_element_type=jnp.float32)
    s = mm2 - esq_ref[...]

    tmax = jnp.max(s, axis=1, keepdims=True)
    cidx = lax.broadcasted_iota(jnp.int32, (MB, NB), 1) + j * NB
    targ = jnp.min(jnp.where(s == tmax, cidx, jnp.int32(2**30)),
                   axis=1, keepdims=True)
    better = tmax > rmax[...]
    ridx[...] = jnp.where(better, targ, ridx[...])
    rmax[...] = jnp.where(better, tmax, rmax[...])

    @pl.when(j == NN - 1)
    def _():
        xsq = jnp.sum(x * x, axis=1, keepdims=True)
        idx_ref[0, 0, :] = ridx[:, 0]
        dmin_ref[0, 0, :] = (xsq - rmax[...])[:, 0]
        bins = lax.broadcasted_iota(jnp.int32, (MB, NUM_EMB), 1)
        onehot = jnp.where(ridx[...] == bins, jnp.int32(1), jnp.int32(0))
        cnt_ref[0] = jnp.sum(onehot, axis=0, keepdims=True)


_argmin_call = pl.pallas_call(
    _argmin_body,
    grid=(NM, NN),
    in_specs=[
        pl.BlockSpec((MB, DIM), lambda i, j: (i, 0)),
        pl.BlockSpec((NB, DIM), lambda i, j: (j, 0)),
        pl.BlockSpec((1, NB), lambda i, j: (0, j)),
    ],
    out_specs=[
        pl.BlockSpec((1, 1, MB), lambda i, j: (i, 0, 0)),
        pl.BlockSpec((1, 1, MB), lambda i, j: (i, 0, 0)),
        pl.BlockSpec((1, 1, NUM_EMB), lambda i, j: (i, 0, 0)),
    ],
    out_shape=[
        jax.ShapeDtypeStruct((NM, 1, MB), jnp.int32),
        jax.ShapeDtypeStruct((NM, 1, MB), jnp.float32),
        jax.ShapeDtypeStruct((NM, 1, NUM_EMB), jnp.int32),
    ],
    scratch_shapes=[
        pltpu.VMEM((MB, 1), jnp.float32),
        pltpu.VMEM((MB, 1), jnp.int32),
    ],
    compiler_params=pltpu.CompilerParams(
        dimension_semantics=("parallel", "arbitrary")),
)


@functools.lru_cache(maxsize=1)
def _sc_gather_call():
    from jax.experimental.pallas import tpu_sc as plsc

    mesh = plsc.VectorSubcoreMesh(core_axis_name="c", subcore_axis_name="s")

    @functools.partial(
        pl.kernel, mesh=mesh,
        out_type=jax.ShapeDtypeStruct((TOKENS, DIM), jnp.float32),
        scratch_types=[
            pltpu.VMEM((256,), jnp.int32),
            pltpu.VMEM((256, DIM), jnp.float32),
            pltpu.SemaphoreType.DMA,
        ],
    )
    def sc_body(w_hbm, idx_hbm, quant_hbm, idx_g, rows_v, sem):
        c = lax.axis_index("c")
        s = lax.axis_index("s")
        wid = s * 2 + c
        base = wid * BPW

        for chunk in range(2):
            off = base + chunk * 256
            pltpu.sync_copy(idx_hbm.at[pl.ds(off, 256)], idx_g)
            pltpu.async_copy(w_hbm.at[idx_g], rows_v, sem).wait()
            pltpu.sync_copy(rows_v, quant_hbm.at[pl.ds(off, 256)])

    return sc_body


def _scalar_body(dmin_ref, counts_ref, loss_ref, perp_ref):
    dmin = dmin_ref[...]
    loss_ref[0, 0] = 1.25 * jnp.sum(dmin) / (float(TOKENS) * float(DIM))
    csum = jnp.sum(counts_ref[...], axis=0)
    avg = csum.astype(jnp.float32) * (1.0 / float(TOKENS))
    plog = avg * jnp.log(avg + 1e-10)
    perp_ref[0, 0] = jnp.exp(-jnp.sum(plog))


_scalar_call = pl.pallas_call(
    _scalar_body,
    in_specs=[
        pl.BlockSpec((NM, 1, MB), lambda: (0, 0, 0)),
        pl.BlockSpec((NM, 1, NUM_EMB), lambda: (0, 0, 0)),
    ],
    out_specs=[
        pl.BlockSpec(memory_space=pltpu.SMEM),
        pl.BlockSpec(memory_space=pltpu.SMEM),
    ],
    out_shape=[
        jax.ShapeDtypeStruct((1, 1), jnp.float32),
        jax.ShapeDtypeStruct((1, 1), jnp.float32),
    ],
)


def kernel(inputs, embedding_weight):
    input_shape = inputs.shape
    flat = inputs.reshape(-1, DIM)
    esq = _esq_call(embedding_weight)
    idx3, dmin3, cnt3 = _argmin_call(flat, embedding_weight, esq)
    idx = idx3.reshape(-1)

    quant = _sc_gather_call()(embedding_weight, idx)
    loss2, perp2 = _scalar_call(dmin3, cnt3)

    quant = quant.reshape(input_shape)
    quantized_st = inputs + (quant - inputs)
    return (loss2[0, 0], quantized_st, perp2[0, 0],
            idx.reshape(input_shape[0], -1))


# speedup vs baseline: 1.0578x; 1.0479x over previous
import functools

import jax
import jax.numpy as jnp
from jax import lax
from jax.experimental import pallas as pl
from jax.experimental.pallas import tpu as pltpu

NUM_EMB = 8192
DIM = 256
TOKENS = 16384
MB = 256
NB = 1024
NM = TOKENS // MB
NN = NUM_EMB // NB

NW = 32
BPW = TOKENS // NW


def _esq_body(e_ref, esq_ref):
    e = e_ref[...]
    esq_ref[...] = jnp.sum(e * e, axis=1).reshape(1, NB)


_esq_call = pl.pallas_call(
    _esq_body,
    grid=(NN,),
    in_specs=[pl.BlockSpec((NB, DIM), lambda j: (j, 0))],
    out_specs=pl.BlockSpec((1, NB), lambda j: (0, j)),
    out_shape=jax.ShapeDtypeStruct((1, NUM_EMB), jnp.float32),
)


def _argmin_body(x_ref, e_ref, esq_ref, idx_ref, dmin_ref, cnt_ref, rmax, ridx):
    j = pl.program_id(1)

    @pl.when(j == 0)
    def _():
        rmax[...] = jnp.full((MB, 1), -jnp.inf, jnp.float32)
        ridx[...] = jnp.zeros((MB, 1), jnp.float32)

    x = x_ref[...]
    e = e_ref[...]

    x2 = (x + x).astype(jnp.bfloat16)
    mm2 = lax.dot_general(x2, e.astype(jnp.bfloat16), (((1,), (1,)), ((), ())),
                          preferred_element_type=jnp.float32)
    s = mm2 - esq_ref[...]

    tmax = jnp.max(s, axis=1, keepdims=True)
    cidx = (lax.broadcasted_iota(jnp.int32, (MB, NB), 1).astype(jnp.float32)
            + jnp.float32(j * NB))
    targ = jnp.min(jnp.where(s == tmax, cidx, jnp.float32(1e9)),
                   axis=1, keepdims=True)
    better = tmax > rmax[...]
    ridx[...] = jnp.where(better, targ, ridx[...])
    rmax[...] = jnp.where(better, tmax, rmax[...])

    @pl.when(j == NN - 1)
    def _():
        xsq = jnp.sum(x * x, axis=1, keepdims=True)
        idx_ref[0, 0, :] = ridx[:, 0].astype(jnp.int32)
        dmin_ref[0, 0, :] = (xsq - rmax[...])[:, 0]
        bins = lax.broadcasted_iota(jnp.int32, (MB, NUM_EMB), 1).astype(jnp.float32)
        onehot = jnp.where(ridx[...] == bins, jnp.int32(1), jnp.int32(0))
        cnt_ref[0] = jnp.sum(onehot, axis=0, keepdims=True)


_argmin_call = pl.pallas_call(
    _argmin_body,
    grid=(NM, NN),
    in_specs=[
        pl.BlockSpec((MB, DIM), lambda i, j: (i, 0)),
        pl.BlockSpec((NB, DIM), lambda i, j: (j, 0)),
        pl.BlockSpec((1, NB), lambda i, j: (0, j)),
    ],
    out_specs=[
        pl.BlockSpec((1, 1, MB), lambda i, j: (i, 0, 0)),
        pl.BlockSpec((1, 1, MB), lambda i, j: (i, 0, 0)),
        pl.BlockSpec((1, 1, NUM_EMB), lambda i, j: (i, 0, 0)),
    ],
    out_shape=[
        jax.ShapeDtypeStruct((NM, 1, MB), jnp.int32),
        jax.ShapeDtypeStruct((NM, 1, MB), jnp.float32),
        jax.ShapeDtypeStruct((NM, 1, NUM_EMB), jnp.int32),
    ],
    scratch_shapes=[
        pltpu.VMEM((MB, 1), jnp.float32),
        pltpu.VMEM((MB, 1), jnp.float32),
    ],
    compiler_params=pltpu.CompilerParams(
        dimension_semantics=("parallel", "arbitrary")),
)


@functools.lru_cache(maxsize=1)
def _sc_gather_call():
    from jax.experimental.pallas import tpu_sc as plsc

    mesh = plsc.VectorSubcoreMesh(core_axis_name="c", subcore_axis_name="s")

    @functools.partial(
        pl.kernel, mesh=mesh,
        out_type=jax.ShapeDtypeStruct((TOKENS, DIM), jnp.float32),
        scratch_types=[
            pltpu.VMEM((256,), jnp.int32),
            pltpu.VMEM((256, DIM), jnp.float32),
            pltpu.SemaphoreType.DMA,
        ],
    )
    def sc_body(w_hbm, idx_hbm, quant_hbm, idx_g, rows_v, sem):
        c = lax.axis_index("c")
        s = lax.axis_index("s")
        wid = s * 2 + c
        base = wid * BPW

        for chunk in range(2):
            off = base + chunk * 256
            pltpu.sync_copy(idx_hbm.at[pl.ds(off, 256)], idx_g)
            pltpu.async_copy(w_hbm.at[idx_g], rows_v, sem).wait()
            pltpu.sync_copy(rows_v, quant_hbm.at[pl.ds(off, 256)])

    return sc_body


def _scalar_body(dmin_ref, counts_ref, loss_ref, perp_ref):
    dmin = dmin_ref[...]
    loss_ref[0, 0] = 1.25 * jnp.sum(dmin) / (float(TOKENS) * float(DIM))
    csum = jnp.sum(counts_ref[...], axis=0)
    avg = csum.astype(jnp.float32) * (1.0 / float(TOKENS))
    plog = avg * jnp.log(avg + 1e-10)
    perp_ref[0, 0] = jnp.exp(-jnp.sum(plog))


_scalar_call = pl.pallas_call(
    _scalar_body,
    in_specs=[
        pl.BlockSpec((NM, 1, MB), lambda: (0, 0, 0)),
        pl.BlockSpec((NM, 1, NUM_EMB), lambda: (0, 0, 0)),
    ],
    out_specs=[
        pl.BlockSpec(memory_space=pltpu.SMEM),
        pl.BlockSpec(memory_space=pltpu.SMEM),
    ],
    out_shape=[
        jax.ShapeDtypeStruct((1, 1), jnp.float32),
        jax.ShapeDtypeStruct((1, 1), jnp.float32),
    ],
)


def kernel(inputs, embedding_weight):
    input_shape = inputs.shape
    flat = inputs.reshape(-1, DIM)
    esq = _esq_call(embedding_weight)
    idx3, dmin3, cnt3 = _argmin_call(flat, embedding_weight, esq)
    idx = idx3.reshape(-1)

    quant = _sc_gather_call()(embedding_weight, idx)
    loss2, perp2 = _scalar_call(dmin3, cnt3)

    quant = quant.reshape(input_shape)
    quantized_st = inputs + (quant - inputs)
    return (loss2[0, 0], quantized_st, perp2[0, 0],
            idx.reshape(input_shape[0], -1))


# speedup vs baseline: 1.5840x; 1.4974x over previous
import functools

import jax
import jax.numpy as jnp
from jax import lax
from jax.experimental import pallas as pl
from jax.experimental.pallas import tpu as pltpu

NUM_EMB = 8192
DIM = 256
TOKENS = 16384
MB = 512
NB = 1024
NM = TOKENS // MB
NN = NUM_EMB // NB

NW = 32
BPW = TOKENS // NW


def _esq_body(e_ref, esq_ref):
    e = e_ref[...]
    esq_ref[...] = jnp.sum(e * e, axis=1).reshape(1, NB)


_esq_call = pl.pallas_call(
    _esq_body,
    grid=(NN,),
    in_specs=[pl.BlockSpec((NB, DIM), lambda j: (j, 0))],
    out_specs=pl.BlockSpec((1, NB), lambda j: (0, j)),
    out_shape=jax.ShapeDtypeStruct((1, NUM_EMB), jnp.float32),
)


def _argmin_body(x_ref, e_ref, esq_ref, idx_ref, dmin_ref, cnt_ref, rmax, ridx):
    j = pl.program_id(1)

    @pl.when(j == 0)
    def _():
        rmax[...] = jnp.full((MB, 1), -jnp.inf, jnp.float32)
        ridx[...] = jnp.zeros((MB, 1), jnp.float32)

    x = x_ref[...]
    e = e_ref[...]

    x2 = (x + x).astype(jnp.bfloat16)
    mm2 = lax.dot_general(x2, e.astype(jnp.bfloat16), (((1,), (1,)), ((), ())),
                          preferred_element_type=jnp.float32)
    s = mm2 - esq_ref[...]

    tmax = jnp.max(s, axis=1, keepdims=True)
    cidx = (lax.broadcasted_iota(jnp.int32, (MB, NB), 1).astype(jnp.float32)
            + jnp.float32(j * NB))
    targ = jnp.min(jnp.where(s == tmax, cidx, jnp.float32(1e9)),
                   axis=1, keepdims=True)
    better = tmax > rmax[...]
    ridx[...] = jnp.where(better, targ, ridx[...])
    rmax[...] = jnp.where(better, tmax, rmax[...])

    @pl.when(j == NN - 1)
    def _():
        xsq = jnp.sum(x * x, axis=1, keepdims=True)
        idx_ref[0, 0, :] = ridx[:, 0].astype(jnp.int32)
        dmin_ref[0, 0, :] = (xsq - rmax[...])[:, 0]
        bins = lax.broadcasted_iota(jnp.int32, (MB, NUM_EMB), 1).astype(jnp.float32)
        onehot = jnp.where(ridx[...] == bins, jnp.int32(1), jnp.int32(0))
        cnt_ref[0] = jnp.sum(onehot, axis=0, keepdims=True)


_argmin_call = pl.pallas_call(
    _argmin_body,
    grid=(NM, NN),
    in_specs=[
        pl.BlockSpec((MB, DIM), lambda i, j: (i, 0)),
        pl.BlockSpec((NB, DIM), lambda i, j: (j, 0)),
        pl.BlockSpec((1, NB), lambda i, j: (0, j)),
    ],
    out_specs=[
        pl.BlockSpec((1, 1, MB), lambda i, j: (i, 0, 0)),
        pl.BlockSpec((1, 1, MB), lambda i, j: (i, 0, 0)),
        pl.BlockSpec((1, 1, NUM_EMB), lambda i, j: (i, 0, 0)),
    ],
    out_shape=[
        jax.ShapeDtypeStruct((NM, 1, MB), jnp.int32),
        jax.ShapeDtypeStruct((NM, 1, MB), jnp.float32),
        jax.ShapeDtypeStruct((NM, 1, NUM_EMB), jnp.int32),
    ],
    scratch_shapes=[
        pltpu.VMEM((MB, 1), jnp.float32),
        pltpu.VMEM((MB, 1), jnp.float32),
    ],
    compiler_params=pltpu.CompilerParams(
        dimension_semantics=("parallel", "arbitrary")),
)


@functools.lru_cache(maxsize=1)
def _sc_gather_call():
    from jax.experimental.pallas import tpu_sc as plsc

    mesh = plsc.VectorSubcoreMesh(core_axis_name="c", subcore_axis_name="s")

    @functools.partial(
        pl.kernel, mesh=mesh,
        out_type=jax.ShapeDtypeStruct((TOKENS, DIM), jnp.float32),
        scratch_types=[
            pltpu.VMEM((256,), jnp.int32),
            pltpu.VMEM((256, DIM), jnp.float32),
            pltpu.SemaphoreType.DMA,
        ],
    )
    def sc_body(w_hbm, idx_hbm, quant_hbm, idx_g, rows_v, sem):
        c = lax.axis_index("c")
        s = lax.axis_index("s")
        wid = s * 2 + c
        base = wid * BPW

        for chunk in range(2):
            off = base + chunk * 256
            pltpu.sync_copy(idx_hbm.at[pl.ds(off, 256)], idx_g)
            pltpu.async_copy(w_hbm.at[idx_g], rows_v, sem).wait()
            pltpu.sync_copy(rows_v, quant_hbm.at[pl.ds(off, 256)])

    return sc_body


def _scalar_body(dmin_ref, counts_ref, loss_ref, perp_ref):
    dmin = dmin_ref[...]
    loss_ref[0, 0] = 1.25 * jnp.sum(dmin) / (float(TOKENS) * float(DIM))
    csum = jnp.sum(counts_ref[...], axis=0)
    avg = csum.astype(jnp.float32) * (1.0 / float(TOKENS))
    plog = avg * jnp.log(avg + 1e-10)
    perp_ref[0, 0] = jnp.exp(-jnp.sum(plog))


_scalar_call = pl.pallas_call(
    _scalar_body,
    in_specs=[
        pl.BlockSpec((NM, 1, MB), lambda: (0, 0, 0)),
        pl.BlockSpec((NM, 1, NUM_EMB), lambda: (0, 0, 0)),
    ],
    out_specs=[
        pl.BlockSpec(memory_space=pltpu.SMEM),
        pl.BlockSpec(memory_space=pltpu.SMEM),
    ],
    out_shape=[
        jax.ShapeDtypeStruct((1, 1), jnp.float32),
        jax.ShapeDtypeStruct((1, 1), jnp.float32),
    ],
)


def kernel(inputs, embedding_weight):
    input_shape = inputs.shape
    flat = inputs.reshape(-1, DIM)
    esq = _esq_call(embedding_weight)
    idx3, dmin3, cnt3 = _argmin_call(flat, embedding_weight, esq)
    idx = idx3.reshape(-1)

    quant = _sc_gather_call()(embedding_weight, idx)
    loss2, perp2 = _scalar_call(dmin3, cnt3)

    quant = quant.reshape(input_shape)
    quantized_st = inputs + (quant - inputs)
    return (loss2[0, 0], quantized_st, perp2[0, 0],
            idx.reshape(input_shape[0], -1))


# speedup vs baseline: 2.0573x; 1.2988x over previous
import functools

import jax
import jax.numpy as jnp
from jax import lax
from jax.experimental import pallas as pl
from jax.experimental.pallas import tpu as pltpu

NUM_EMB = 8192
DIM = 256
TOKENS = 16384
MB = 1024
NB = 1024
NM = TOKENS // MB
NN = NUM_EMB // NB

NW = 32
BPW = TOKENS // NW


def _esq_body(e_ref, esq_ref):
    e = e_ref[...]
    esq_ref[...] = jnp.sum(e * e, axis=1).reshape(1, NB)


_esq_call = pl.pallas_call(
    _esq_body,
    grid=(NN,),
    in_specs=[pl.BlockSpec((NB, DIM), lambda j: (j, 0))],
    out_specs=pl.BlockSpec((1, NB), lambda j: (0, j)),
    out_shape=jax.ShapeDtypeStruct((1, NUM_EMB), jnp.float32),
)


def _argmin_body(x_ref, e_ref, esq_ref, idx_ref, dmin_ref, cnt_ref, rmax, ridx):
    j = pl.program_id(1)

    @pl.when(j == 0)
    def _():
        rmax[...] = jnp.full((MB, 1), -jnp.inf, jnp.float32)
        ridx[...] = jnp.zeros((MB, 1), jnp.float32)

    x = x_ref[...]
    e = e_ref[...]

    x2 = (x + x).astype(jnp.bfloat16)
    mm2 = lax.dot_general(x2, e.astype(jnp.bfloat16), (((1,), (1,)), ((), ())),
                          preferred_element_type=jnp.float32)
    s = mm2 - esq_ref[...]

    tmax = jnp.max(s, axis=1, keepdims=True)
    cidx = (lax.broadcasted_iota(jnp.int32, (MB, NB), 1).astype(jnp.float32)
            + jnp.float32(j * NB))
    targ = jnp.min(jnp.where(s == tmax, cidx, jnp.float32(1e9)),
                   axis=1, keepdims=True)
    better = tmax > rmax[...]
    ridx[...] = jnp.where(better, targ, ridx[...])
    rmax[...] = jnp.where(better, tmax, rmax[...])

    @pl.when(j == NN - 1)
    def _():
        xsq = jnp.sum(x * x, axis=1, keepdims=True)
        idx_ref[0, 0, :] = ridx[:, 0].astype(jnp.int32)
        dmin_ref[0, 0, :] = (xsq - rmax[...])[:, 0]
        bins = lax.broadcasted_iota(jnp.int32, (MB, NUM_EMB), 1).astype(jnp.float32)
        onehot = jnp.where(ridx[...] == bins, jnp.int32(1), jnp.int32(0))
        cnt_ref[0] = jnp.sum(onehot, axis=0, keepdims=True)


_argmin_call = pl.pallas_call(
    _argmin_body,
    grid=(NM, NN),
    in_specs=[
        pl.BlockSpec((MB, DIM), lambda i, j: (i, 0)),
        pl.BlockSpec((NB, DIM), lambda i, j: (j, 0)),
        pl.BlockSpec((1, NB), lambda i, j: (0, j)),
    ],
    out_specs=[
        pl.BlockSpec((1, 1, MB), lambda i, j: (i, 0, 0)),
        pl.BlockSpec((1, 1, MB), lambda i, j: (i, 0, 0)),
        pl.BlockSpec((1, 1, NUM_EMB), lambda i, j: (i, 0, 0)),
    ],
    out_shape=[
        jax.ShapeDtypeStruct((NM, 1, MB), jnp.int32),
        jax.ShapeDtypeStruct((NM, 1, MB), jnp.float32),
        jax.ShapeDtypeStruct((NM, 1, NUM_EMB), jnp.int32),
    ],
    scratch_shapes=[
        pltpu.VMEM((MB, 1), jnp.float32),
        pltpu.VMEM((MB, 1), jnp.float32),
    ],
    compiler_params=pltpu.CompilerParams(
        dimension_semantics=("parallel", "arbitrary")),
)


@functools.lru_cache(maxsize=1)
def _sc_gather_call():
    from jax.experimental.pallas import tpu_sc as plsc

    mesh = plsc.VectorSubcoreMesh(core_axis_name="c", subcore_axis_name="s")

    @functools.partial(
        pl.kernel, mesh=mesh,
        out_type=jax.ShapeDtypeStruct((TOKENS, DIM), jnp.float32),
        scratch_types=[
            pltpu.VMEM((256,), jnp.int32),
            pltpu.VMEM((256, DIM), jnp.float32),
            pltpu.SemaphoreType.DMA,
        ],
    )
    def sc_body(w_hbm, idx_hbm, quant_hbm, idx_g, rows_v, sem):
        c = lax.axis_index("c")
        s = lax.axis_index("s")
        wid = s * 2 + c
        base = wid * BPW

        for chunk in range(2):
            off = base + chunk * 256
            pltpu.sync_copy(idx_hbm.at[pl.ds(off, 256)], idx_g)
            pltpu.async_copy(w_hbm.at[idx_g], rows_v, sem).wait()
            pltpu.sync_copy(rows_v, quant_hbm.at[pl.ds(off, 256)])

    return sc_body


def _scalar_body(dmin_ref, counts_ref, loss_ref, perp_ref):
    dmin = dmin_ref[...]
    loss_ref[0, 0] = 1.25 * jnp.sum(dmin) / (float(TOKENS) * float(DIM))
    csum = jnp.sum(counts_ref[...], axis=0)
    avg = csum.astype(jnp.float32) * (1.0 / float(TOKENS))
    plog = avg * jnp.log(avg + 1e-10)
    perp_ref[0, 0] = jnp.exp(-jnp.sum(plog))


_scalar_call = pl.pallas_call(
    _scalar_body,
    in_specs=[
        pl.BlockSpec((NM, 1, MB), lambda: (0, 0, 0)),
        pl.BlockSpec((NM, 1, NUM_EMB), lambda: (0, 0, 0)),
    ],
    out_specs=[
        pl.BlockSpec(memory_space=pltpu.SMEM),
        pl.BlockSpec(memory_space=pltpu.SMEM),
    ],
    out_shape=[
        jax.ShapeDtypeStruct((1, 1), jnp.float32),
        jax.ShapeDtypeStruct((1, 1), jnp.float32),
    ],
)


def kernel(inputs, embedding_weight):
    input_shape = inputs.shape
    flat = inputs.reshape(-1, DIM)
    esq = _esq_call(embedding_weight)
    idx3, dmin3, cnt3 = _argmin_call(flat, embedding_weight, esq)
    idx = idx3.reshape(-1)

    quant = _sc_gather_call()(embedding_weight, idx)
    loss2, perp2 = _scalar_call(dmin3, cnt3)

    quant = quant.reshape(input_shape)
    quantized_st = inputs + (quant - inputs)
    return (loss2[0, 0], quantized_st, perp2[0, 0],
            idx.reshape(input_shape[0], -1))


# speedup vs baseline: 2.2573x; 1.0972x over previous
import functools

import jax
import jax.numpy as jnp
from jax import lax
from jax.experimental import pallas as pl
from jax.experimental.pallas import tpu as pltpu

NUM_EMB = 8192
DIM = 256
TOKENS = 16384
MB = 2048
NB = 1024
NM = TOKENS // MB
NN = NUM_EMB // NB

NW = 32
BPW = TOKENS // NW


def _esq_body(e_ref, esq_ref):
    e = e_ref[...]
    esq_ref[...] = jnp.sum(e * e, axis=1).reshape(1, NB)


_esq_call = pl.pallas_call(
    _esq_body,
    grid=(NN,),
    in_specs=[pl.BlockSpec((NB, DIM), lambda j: (j, 0))],
    out_specs=pl.BlockSpec((1, NB), lambda j: (0, j)),
    out_shape=jax.ShapeDtypeStruct((1, NUM_EMB), jnp.float32),
)


def _argmin_body(x_ref, e_ref, esq_ref, idx_ref, dmin_ref, cnt_ref, rmax, ridx):
    j = pl.program_id(1)

    @pl.when(j == 0)
    def _():
        rmax[...] = jnp.full((MB, 1), -jnp.inf, jnp.float32)
        ridx[...] = jnp.zeros((MB, 1), jnp.float32)

    x = x_ref[...]
    e = e_ref[...]

    x2 = (x + x).astype(jnp.bfloat16)
    mm2 = lax.dot_general(x2, e.astype(jnp.bfloat16), (((1,), (1,)), ((), ())),
                          preferred_element_type=jnp.float32)
    s = mm2 - esq_ref[...]

    tmax = jnp.max(s, axis=1, keepdims=True)
    cidx = (lax.broadcasted_iota(jnp.int32, (MB, NB), 1).astype(jnp.float32)
            + jnp.float32(j * NB))
    targ = jnp.min(jnp.where(s == tmax, cidx, jnp.float32(1e9)),
                   axis=1, keepdims=True)
    better = tmax > rmax[...]
    ridx[...] = jnp.where(better, targ, ridx[...])
    rmax[...] = jnp.where(better, tmax, rmax[...])

    @pl.when(j == NN - 1)
    def _():
        xsq = jnp.sum(x * x, axis=1, keepdims=True)
        idx_ref[0, 0, :] = ridx[:, 0].astype(jnp.int32)
        dmin_ref[0, 0, :] = (xsq - rmax[...])[:, 0]
        bins = lax.broadcasted_iota(jnp.int32, (MB, NUM_EMB), 1).astype(jnp.float32)
        onehot = jnp.where(ridx[...] == bins, jnp.int32(1), jnp.int32(0))
        cnt_ref[0] = jnp.sum(onehot, axis=0, keepdims=True)


_argmin_call = pl.pallas_call(
    _argmin_body,
    grid=(NM, NN),
    in_specs=[
        pl.BlockSpec((MB, DIM), lambda i, j: (i, 0)),
        pl.BlockSpec((NB, DIM), lambda i, j: (j, 0)),
        pl.BlockSpec((1, NB), lambda i, j: (0, j)),
    ],
    out_specs=[
        pl.BlockSpec((1, 1, MB), lambda i, j: (i, 0, 0)),
        pl.BlockSpec((1, 1, MB), lambda i, j: (i, 0, 0)),
        pl.BlockSpec((1, 1, NUM_EMB), lambda i, j: (i, 0, 0)),
    ],
    out_shape=[
        jax.ShapeDtypeStruct((NM, 1, MB), jnp.int32),
        jax.ShapeDtypeStruct((NM, 1, MB), jnp.float32),
        jax.ShapeDtypeStruct((NM, 1, NUM_EMB), jnp.int32),
    ],
    scratch_shapes=[
        pltpu.VMEM((MB, 1), jnp.float32),
        pltpu.VMEM((MB, 1), jnp.float32),
    ],
    compiler_params=pltpu.CompilerParams(
        dimension_semantics=("parallel", "arbitrary")),
)


@functools.lru_cache(maxsize=1)
def _sc_gather_call():
    from jax.experimental.pallas import tpu_sc as plsc

    mesh = plsc.VectorSubcoreMesh(core_axis_name="c", subcore_axis_name="s")

    @functools.partial(
        pl.kernel, mesh=mesh,
        out_type=jax.ShapeDtypeStruct((TOKENS, DIM), jnp.float32),
        scratch_types=[
            pltpu.VMEM((256,), jnp.int32),
            pltpu.VMEM((256, DIM), jnp.float32),
            pltpu.SemaphoreType.DMA,
        ],
    )
    def sc_body(w_hbm, idx_hbm, quant_hbm, idx_g, rows_v, sem):
        c = lax.axis_index("c")
        s = lax.axis_index("s")
        wid = s * 2 + c
        base = wid * BPW

        for chunk in range(2):
            off = base + chunk * 256
            pltpu.sync_copy(idx_hbm.at[pl.ds(off, 256)], idx_g)
            pltpu.async_copy(w_hbm.at[idx_g], rows_v, sem).wait()
            pltpu.sync_copy(rows_v, quant_hbm.at[pl.ds(off, 256)])

    return sc_body


def _scalar_body(dmin_ref, counts_ref, loss_ref, perp_ref):
    dmin = dmin_ref[...]
    loss_ref[0, 0] = 1.25 * jnp.sum(dmin) / (float(TOKENS) * float(DIM))
    csum = jnp.sum(counts_ref[...], axis=0)
    avg = csum.astype(jnp.float32) * (1.0 / float(TOKENS))
    plog = avg * jnp.log(avg + 1e-10)
    perp_ref[0, 0] = jnp.exp(-jnp.sum(plog))


_scalar_call = pl.pallas_call(
    _scalar_body,
    in_specs=[
        pl.BlockSpec((NM, 1, MB), lambda: (0, 0, 0)),
        pl.BlockSpec((NM, 1, NUM_EMB), lambda: (0, 0, 0)),
    ],
    out_specs=[
        pl.BlockSpec(memory_space=pltpu.SMEM),
        pl.BlockSpec(memory_space=pltpu.SMEM),
    ],
    out_shape=[
        jax.ShapeDtypeStruct((1, 1), jnp.float32),
        jax.ShapeDtypeStruct((1, 1), jnp.float32),
    ],
)


def kernel(inputs, embedding_weight):
    input_shape = inputs.shape
    flat = inputs.reshape(-1, DIM)
    esq = _esq_call(embedding_weight)
    idx3, dmin3, cnt3 = _argmin_call(flat, embedding_weight, esq)
    idx = idx3.reshape(-1)

    quant = _sc_gather_call()(embedding_weight, idx)
    loss2, perp2 = _scalar_call(dmin3, cnt3)

    quant = quant.reshape(input_shape)
    quantized_st = inputs + (quant - inputs)
    return (loss2[0, 0], quantized_st, perp2[0, 0],
            idx.reshape(input_shape[0], -1))
